# baseline (speedup 1.0000x reference)
import jax
import jax.numpy as jnp
from jax.experimental import pallas as pl
from jax.experimental.pallas import tpu as pltpu

B, N, S, D, P = 4, 8, 8192, 64, 64
SCHUNK = 8192
NJ = S // SCHUNK


def _router_body(hs_ref, w8_ref, bc_ref, br_ref, out_ref, acc_ref):
    bi = pl.program_id(0)
    j = pl.program_id(1)

    @pl.when(jnp.logical_and(bi == 0, j == 0))
    def _init():
        acc_ref[...] = jnp.zeros_like(acc_ref)

    x_all = hs_ref[0].reshape(N * D, SCHUNK)
    y = jax.lax.dot_general(
        w8_ref[...], x_all, (((1,), (0,)), ((), ())),
        preferred_element_type=jnp.float32,
    )
    t = y + jnp.float32(N) * bc_ref[...]
    cnt = jnp.sum((t > 0).astype(jnp.float32), axis=1)

    row = jax.lax.broadcasted_iota(jnp.int32, (8, P), 0)
    acc_ref[...] += jnp.where(row == bi, cnt[None, :], 0.0)

    @pl.when(jnp.logical_and(bi == B - 1, j == NJ - 1))
    def _finish():
        g = acc_ref[0:B, :] * jnp.float32(1.0 / S)
        z = jax.lax.dot_general(
            g, w8_ref[:, 0:D], (((1,), (1,)), ((), ())),
            preferred_element_type=jnp.float32,
        ) + br_ref[...]
        m = jnp.max(z, axis=1, keepdims=True)
        ii = jax.lax.broadcasted_iota(jnp.int32, (B, P), 1)
        idx = jnp.min(jnp.where(z == m, ii, P), axis=1, keepdims=True)
        out_ref[...] = (ii == idx).astype(jnp.int32)


def kernel(hidden_states, W, b):
    hst = hidden_states.transpose(0, 1, 3, 2)
    w8 = jnp.tile(W, (1, N))
    bc = b.reshape(P, 1)
    br = b.reshape(1, P)
    return pl.pallas_call(
        _router_body,
        grid=(B, NJ),
        in_specs=[
            pl.BlockSpec((1, N, D, SCHUNK), lambda bi, j: (bi, 0, 0, j)),
            pl.BlockSpec((P, N * D), lambda bi, j: (0, 0)),
            pl.BlockSpec((P, 1), lambda bi, j: (0, 0)),
            pl.BlockSpec((1, P), lambda bi, j: (0, 0)),
        ],
        out_specs=pl.BlockSpec((B, P), lambda bi, j: (0, 0)),
        out_shape=jax.ShapeDtypeStruct((B, P), jnp.int32),
        scratch_shapes=[pltpu.VMEM((8, P), jnp.float32)],
    )(hst, w8, bc, br)


# speedup vs baseline: 1.0634x; 1.0634x over previous
import jax
import jax.numpy as jnp
from jax.experimental import pallas as pl
from jax.experimental.pallas import tpu as pltpu

B, N, S, D, P = 4, 8, 8192, 64, 64
SCHUNK = 4096
NJ = S // SCHUNK


def _router_body(hs_ref, w8_ref, bc_ref, br_ref, out_ref, acc_ref):
    bi = pl.program_id(0)
    j = pl.program_id(1)

    @pl.when(jnp.logical_and(bi == 0, j == 0))
    def _init():
        acc_ref[...] = jnp.zeros_like(acc_ref)

    x_all = hs_ref[0].reshape(N * D, SCHUNK)
    y = jax.lax.dot_general(
        w8_ref[...], x_all, (((1,), (0,)), ((), ())),
        preferred_element_type=jnp.float32,
    )
    t = y + jnp.float32(N) * bc_ref[...]
    cnt = jnp.sum((t > 0).astype(jnp.float32), axis=1)

    row = jax.lax.broadcasted_iota(jnp.int32, (8, P), 0)
    acc_ref[...] += jnp.where(row == bi, cnt[None, :], 0.0)

    @pl.when(jnp.logical_and(bi == B - 1, j == NJ - 1))
    def _finish():
        g = acc_ref[0:B, :] * jnp.float32(1.0 / S)
        z = jax.lax.dot_general(
            g, w8_ref[:, 0:D], (((1,), (1,)), ((), ())),
            preferred_element_type=jnp.float32,
        ) + br_ref[...]
        m = jnp.max(z, axis=1, keepdims=True)
        ii = jax.lax.broadcasted_iota(jnp.int32, (B, P), 1)
        idx = jnp.min(jnp.where(z == m, ii, P), axis=1, keepdims=True)
        out_ref[...] = (ii == idx).astype(jnp.int32)


def kernel(hidden_states, W, b):
    hst = hidden_states.transpose(0, 1, 3, 2)
    w8 = jnp.tile(W, (1, N))
    bc = b.reshape(P, 1)
    br = b.reshape(1, P)
    return pl.pallas_call(
        _router_body,
        grid=(B, NJ),
        in_specs=[
            pl.BlockSpec((1, N, D, SCHUNK), lambda bi, j: (bi, 0, 0, j)),
            pl.BlockSpec((P, N * D), lambda bi, j: (0, 0)),
            pl.BlockSpec((P, 1), lambda bi, j: (0, 0)),
            pl.BlockSpec((1, P), lambda bi, j: (0, 0)),
        ],
        out_specs=pl.BlockSpec((B, P), lambda bi, j: (0, 0)),
        out_shape=jax.ShapeDtypeStruct((B, P), jnp.int32),
        scratch_shapes=[pltpu.VMEM((8, P), jnp.float32)],
    )(hst, w8, bc, br)
